# baseline (speedup 1.0000x reference)
import jax
import jax.numpy as jnp
from jax import lax
from jax.experimental import pallas as pl
from jax.experimental.pallas import tpu as pltpu
from jax.experimental.pallas import tpu_sc as plsc

_SPCH = 512
_SPK = 2
_SPL = 1
_SP_ROWS = 8704

_TCH = 16
_TK = 3
_TL = 1
_T_ROWS = 512


def _ring(in_copy, out_copy, nch, K, L):
    for i in range(-L, nch):
        if i >= 0:
            in_copy(i).wait()
            out_copy(i).start()
        j = i + L
        if 0 <= j < nch:
            if j >= K:
                out_copy(j - K).wait()
            in_copy(j).start()
    for i in range(max(0, nch - K), nch):
        out_copy(i).wait()


def kernel(x):
    M, N = x.shape
    info = plsc.get_sparse_core_info()
    NC, NS = info.num_cores, info.num_subcores
    rpc = M // NC
    assert _SP_ROWS + (NS - 1) * _T_ROWS == rpc

    vmesh = plsc.VectorSubcoreMesh(core_axis_name="c", subcore_axis_name="s")
    smesh = plsc.ScalarSubcoreMesh(axis_name="c", num_cores=NC)

    def scs_fn(x_hbm, out_hbm, spbufs, tbufs, sp_in, sp_out, t_in, t_out):
        del tbufs, t_in, t_out
        c = lax.axis_index("c")
        base = c * rpc

        def in_copy(i):
            return pltpu.make_async_copy(
                x_hbm.at[pl.ds(base + i * _SPCH, _SPCH), :],
                spbufs.at[i % _SPK],
                sp_in[i % _SPK],
            )

        def out_copy(i):
            return pltpu.make_async_copy(
                spbufs.at[i % _SPK],
                out_hbm.at[pl.ds(base + i * _SPCH, _SPCH), :],
                sp_out[i % _SPK],
            )

        _ring(in_copy, out_copy, _SP_ROWS // _SPCH, _SPK, _SPL)

    def tec_fn(x_hbm, out_hbm, spbufs, tbufs, sp_in, sp_out, t_in, t_out):
        del spbufs, sp_in, sp_out
        c = lax.axis_index("c")
        s = lax.axis_index("s")
        cbase = c * rpc + _SP_ROWS

        @pl.when(s > 0)
        def _():
            tbase = cbase + (s - 1) * _T_ROWS

            def in_copy(i):
                return pltpu.make_async_copy(
                    x_hbm.at[pl.ds(tbase + i * _TCH, _TCH), :],
                    tbufs.at[i % _TK],
                    t_in[i % _TK],
                )

            def out_copy(i):
                return pltpu.make_async_copy(
                    tbufs.at[i % _TK],
                    out_hbm.at[pl.ds(tbase + i * _TCH, _TCH), :],
                    t_out[i % _TK],
                )

            _ring(in_copy, out_copy, _T_ROWS // _TCH, _TK, _TL)

    copy_k = pl.kernel(
        [scs_fn, tec_fn],
        out_type=jax.ShapeDtypeStruct((M, N), x.dtype),
        mesh=[smesh, vmesh],
        scratch_types=[
            pltpu.MemorySpace.VMEM_SHARED((_SPK, _SPCH, N), x.dtype),
            (pltpu.MemorySpace.VMEM @ vmesh)((_TK, _TCH, N), x.dtype),
            [pltpu.SemaphoreType.DMA @ smesh] * _SPK,
            [pltpu.SemaphoreType.DMA @ smesh] * _SPK,
            [pltpu.SemaphoreType.DMA @ vmesh] * _TK,
            [pltpu.SemaphoreType.DMA @ vmesh] * _TK,
        ],
    )

    gathered = copy_k(x)
    sizes = jnp.asarray([M], dtype=jnp.int32)
    return (gathered, sizes)
